# Initial kernel scaffold; baseline (speedup 1.0000x reference)
import jax
import jax.numpy as jnp
from jax.experimental import pallas as pl


def kernel(positions, model):
    raise NotImplementedError("write your pallas kernel here")


# speedup vs baseline: 204.0757x; 204.0757x over previous
import functools

import jax
import jax.numpy as jnp
from jax import lax
from jax.experimental import pallas as pl
from jax.experimental.pallas import tpu as pltpu
from jax.experimental.pallas import tpu_sc as plsc

_BATCH = 16384
_SLATE = 200
_POSITIONS = 200
_N = _BATCH * _SLATE
_NW = 32
_W = _N // _NW
_CHUNK = 20480
_LANES = 16

_mesh = plsc.VectorSubcoreMesh(core_axis_name="c", subcore_axis_name="s")


@functools.partial(
    pl.kernel,
    mesh=_mesh,
    out_type=jax.ShapeDtypeStruct((_N,), jnp.float32),
    scratch_types=[
        pltpu.VMEM((_POSITIONS,), jnp.float32),
        pltpu.VMEM((_CHUNK,), jnp.int32),
        pltpu.VMEM((_CHUNK,), jnp.float32),
    ],
    compiler_params=pltpu.CompilerParams(needs_layout_passes=False),
)
def _gather_kernel(pos_hbm, model_hbm, out_hbm, table_v, idx_v, val_v):
    wid = lax.axis_index("s") * 2 + lax.axis_index("c")
    base = wid * _W
    pltpu.sync_copy(model_hbm, table_v)

    def chunk_body(ci, carry):
        off = base + ci * _CHUNK
        pltpu.sync_copy(pos_hbm.at[pl.ds(off, _CHUNK)], idx_v)

        def gather_body(i, c):
            idx = idx_v[pl.ds(i * _LANES, _LANES)]
            val_v[pl.ds(i * _LANES, _LANES)] = plsc.load_gather(table_v, [idx])
            return c

        lax.fori_loop(0, _CHUNK // _LANES, gather_body, 0, unroll=8)
        pltpu.sync_copy(val_v, out_hbm.at[pl.ds(off, _CHUNK)])
        return carry

    lax.fori_loop(0, _W // _CHUNK, chunk_body, 0)


def kernel(positions, model):
    flat = positions.reshape(_N)
    out = _gather_kernel(flat, model)
    return out.reshape(_BATCH, _SLATE)


# speedup vs baseline: 216.1520x; 1.0592x over previous
import functools

import jax
import jax.numpy as jnp
from jax import lax
from jax.experimental import pallas as pl
from jax.experimental.pallas import tpu as pltpu
from jax.experimental.pallas import tpu_sc as plsc

_BATCH = 16384
_SLATE = 200
_POSITIONS = 200
_N = _BATCH * _SLATE
_NW = 32
_W = _N // _NW
_CHUNK = 12800
_NCHUNK = _W // _CHUNK
_LANES = 16

_mesh = plsc.VectorSubcoreMesh(core_axis_name="c", subcore_axis_name="s")


@functools.partial(
    pl.kernel,
    mesh=_mesh,
    out_type=jax.ShapeDtypeStruct((_N,), jnp.float32),
    scratch_types=[
        pltpu.VMEM((_POSITIONS,), jnp.float32),
        pltpu.VMEM((_CHUNK,), jnp.int32),
        pltpu.VMEM((_CHUNK,), jnp.int32),
        pltpu.VMEM((_CHUNK,), jnp.float32),
        pltpu.VMEM((_CHUNK,), jnp.float32),
        pltpu.SemaphoreType.DMA,
        pltpu.SemaphoreType.DMA,
        pltpu.SemaphoreType.DMA,
        pltpu.SemaphoreType.DMA,
    ],
    compiler_params=pltpu.CompilerParams(needs_layout_passes=False),
)
def _gather_kernel(pos_hbm, model_hbm, out_hbm, table_v,
                   idx0, idx1, val0, val1, sin0, sin1, sout0, sout1):
    wid = lax.axis_index("s") * 2 + lax.axis_index("c")
    base = wid * _W
    pltpu.sync_copy(model_hbm, table_v)

    idx = [idx0, idx1]
    val = [val0, val1]
    sin = [sin0, sin1]
    sout = [sout0, sout1]

    def start_in(g):
        off = base + g * _CHUNK
        return pltpu.async_copy(pos_hbm.at[pl.ds(off, _CHUNK)], idx[g % 2],
                                sin[g % 2])

    def start_out(g):
        off = base + g * _CHUNK
        return pltpu.async_copy(val[g % 2], out_hbm.at[pl.ds(off, _CHUNK)],
                                sout[g % 2])

    in_h = {0: start_in(0), 1: start_in(1)}
    out_h = {}
    for g in range(_NCHUNK):
        b = g % 2
        in_h[g].wait()
        if g >= 2:
            out_h[g - 2].wait()

        def gather_body(i, c, ib=idx[b], vb=val[b]):
            x = ib[pl.ds(i * _LANES, _LANES)]
            vb[pl.ds(i * _LANES, _LANES)] = plsc.load_gather(table_v, [x])
            return c

        lax.fori_loop(0, _CHUNK // _LANES, gather_body, 0, unroll=8)
        out_h[g] = start_out(g)
        if g + 2 < _NCHUNK:
            in_h[g + 2] = start_in(g + 2)
    out_h[_NCHUNK - 2].wait()
    out_h[_NCHUNK - 1].wait()


def kernel(positions, model):
    flat = positions.reshape(_N)
    out = _gather_kernel(flat, model)
    return out.reshape(_BATCH, _SLATE)


# speedup vs baseline: 304.2586x; 1.4076x over previous
import functools

import jax
import jax.numpy as jnp
from jax import lax
from jax.experimental import pallas as pl
from jax.experimental.pallas import tpu as pltpu
from jax.experimental.pallas import tpu_sc as plsc

_BATCH = 16384
_SLATE = 200
_POSITIONS = 200
_N = _BATCH * _SLATE
_NW = 32
_W = _N // _NW
_CHUNK = 12800
_NCHUNK = _W // _CHUNK
_LANES = 16

_mesh = plsc.VectorSubcoreMesh(core_axis_name="c", subcore_axis_name="s")


@functools.partial(
    pl.kernel,
    mesh=_mesh,
    out_type=jax.ShapeDtypeStruct((_N,), jnp.float32),
    scratch_types=[
        pltpu.VMEM((_POSITIONS,), jnp.float32),
        pltpu.VMEM((_CHUNK,), jnp.int32),
        pltpu.VMEM((_CHUNK,), jnp.int32),
        pltpu.VMEM((_CHUNK,), jnp.float32),
        pltpu.VMEM((_CHUNK,), jnp.float32),
        pltpu.SemaphoreType.DMA,
        pltpu.SemaphoreType.DMA,
        pltpu.SemaphoreType.DMA,
        pltpu.SemaphoreType.DMA,
    ],
    compiler_params=pltpu.CompilerParams(needs_layout_passes=False),
)
def _gather_kernel(pos_hbm, model_hbm, out_hbm, table_v,
                   idx0, idx1, val0, val1, sin0, sin1, sout0, sout1):
    wid = lax.axis_index("s") * 2 + lax.axis_index("c")
    base = wid * _W
    pltpu.sync_copy(model_hbm, table_v)

    idx = [idx0, idx1]
    val = [val0, val1]
    sin = [sin0, sin1]
    sout = [sout0, sout1]

    def start_in(g):
        off = base + g * _CHUNK
        return pltpu.async_copy(pos_hbm.at[pl.ds(off, _CHUNK)], idx[g % 2],
                                sin[g % 2])

    def start_out(g):
        off = base + g * _CHUNK
        return pltpu.async_copy(val[g % 2], out_hbm.at[pl.ds(off, _CHUNK)],
                                sout[g % 2])

    in_h = {0: start_in(0), 1: start_in(1)}
    out_h = {}
    for g in range(_NCHUNK):
        b = g % 2
        in_h[g].wait()
        if g >= 2:
            out_h[g - 2].wait()

        ib, vb = idx[b], val[b]

        @plsc.parallel_loop(0, _CHUNK, step=_LANES, unroll=8)
        def _(i, ib=ib, vb=vb):
            x = ib[pl.ds(i, _LANES)]
            vb[pl.ds(i, _LANES)] = plsc.load_gather(table_v, [x])
        out_h[g] = start_out(g)
        if g + 2 < _NCHUNK:
            in_h[g + 2] = start_in(g + 2)
    out_h[_NCHUNK - 2].wait()
    out_h[_NCHUNK - 1].wait()


def kernel(positions, model):
    flat = positions.reshape(_N)
    out = _gather_kernel(flat, model)
    return out.reshape(_BATCH, _SLATE)


# speedup vs baseline: 527.3385x; 1.7332x over previous
import functools

import jax
import jax.numpy as jnp
from jax import lax
from jax.experimental import pallas as pl
from jax.experimental.pallas import tpu as pltpu
from jax.experimental.pallas import tpu_sc as plsc

_BATCH = 16384
_SLATE = 200
_POSITIONS = 200
_NW = 32
_ROWS_PER_W = _BATCH // _NW
_RCHUNK = 64
_NCHUNK = _ROWS_PER_W // _RCHUNK
_LANES = 16
_OFFS = tuple(range(0, _SLATE - _LANES, _LANES)) + (_SLATE - _LANES,)

_mesh = plsc.VectorSubcoreMesh(core_axis_name="c", subcore_axis_name="s")


@functools.partial(
    pl.kernel,
    mesh=_mesh,
    out_type=jax.ShapeDtypeStruct((_BATCH, _SLATE), jnp.float32),
    scratch_types=[
        pltpu.VMEM((_POSITIONS,), jnp.float32),
        pltpu.VMEM((_RCHUNK, _SLATE), jnp.int32),
        pltpu.VMEM((_RCHUNK, _SLATE), jnp.int32),
        pltpu.VMEM((_RCHUNK, _SLATE), jnp.float32),
        pltpu.VMEM((_RCHUNK, _SLATE), jnp.float32),
        pltpu.SemaphoreType.DMA,
        pltpu.SemaphoreType.DMA,
        pltpu.SemaphoreType.DMA,
        pltpu.SemaphoreType.DMA,
    ],
    compiler_params=pltpu.CompilerParams(needs_layout_passes=False),
)
def _gather_kernel(pos_hbm, model_hbm, out_hbm, table_v,
                   idx0, idx1, val0, val1, sin0, sin1, sout0, sout1):
    wid = lax.axis_index("s") * 2 + lax.axis_index("c")
    row0 = wid * _ROWS_PER_W
    pltpu.sync_copy(model_hbm, table_v)

    idx = [idx0, idx1]
    val = [val0, val1]
    sin = [sin0, sin1]
    sout = [sout0, sout1]

    def start_in(g):
        r = row0 + g * _RCHUNK
        return pltpu.async_copy(pos_hbm.at[pl.ds(r, _RCHUNK), :], idx[g % 2],
                                sin[g % 2])

    def start_out(g):
        r = row0 + g * _RCHUNK
        return pltpu.async_copy(val[g % 2], out_hbm.at[pl.ds(r, _RCHUNK), :],
                                sout[g % 2])

    in_h = {0: start_in(0), 1: start_in(1)}
    out_h = {}
    for g in range(_NCHUNK):
        b = g % 2
        in_h[g].wait()
        if g >= 2:
            out_h[g - 2].wait()
        ib, vb = idx[b], val[b]

        @plsc.parallel_loop(0, _RCHUNK, unroll=2)
        def _(r, ib=ib, vb=vb):
            for off in _OFFS:
                x = ib[r, pl.ds(off, _LANES)]
                vb[r, pl.ds(off, _LANES)] = plsc.load_gather(table_v, [x])

        out_h[g] = start_out(g)
        if g + 2 < _NCHUNK:
            in_h[g + 2] = start_in(g + 2)
    out_h[_NCHUNK - 2].wait()
    out_h[_NCHUNK - 1].wait()


def kernel(positions, model):
    return _gather_kernel(positions, model)


# speedup vs baseline: 527.3627x; 1.0000x over previous
import functools

import jax
import jax.numpy as jnp
from jax import lax
from jax.experimental import pallas as pl
from jax.experimental.pallas import tpu as pltpu
from jax.experimental.pallas import tpu_sc as plsc

_BATCH = 16384
_SLATE = 200
_POSITIONS = 200
_NW = 32
_ROWS_PER_W = _BATCH // _NW
_RCHUNK = 64
_NCHUNK = _ROWS_PER_W // _RCHUNK
_LANES = 16
_OFFS = tuple(range(0, _SLATE - _LANES, _LANES)) + (_SLATE - _LANES,)

_mesh = plsc.VectorSubcoreMesh(core_axis_name="c", subcore_axis_name="s")


@functools.partial(
    pl.kernel,
    mesh=_mesh,
    out_type=jax.ShapeDtypeStruct((_BATCH, _SLATE), jnp.float32),
    scratch_types=[
        pltpu.VMEM((_POSITIONS,), jnp.float32),
        pltpu.VMEM((_RCHUNK, _SLATE), jnp.int32),
        pltpu.VMEM((_RCHUNK, _SLATE), jnp.int32),
        pltpu.VMEM((_RCHUNK, _SLATE), jnp.float32),
        pltpu.VMEM((_RCHUNK, _SLATE), jnp.float32),
        pltpu.SemaphoreType.DMA,
        pltpu.SemaphoreType.DMA,
        pltpu.SemaphoreType.DMA,
        pltpu.SemaphoreType.DMA,
    ],
    compiler_params=pltpu.CompilerParams(
        needs_layout_passes=False,
        skip_device_barrier=True,
        disable_bounds_checks=True,
        disable_semaphore_checks=True,
    ),
)
def _gather_kernel(pos_hbm, model_hbm, out_hbm, table_v,
                   idx0, idx1, val0, val1, sin0, sin1, sout0, sout1):
    wid = lax.axis_index("s") * 2 + lax.axis_index("c")
    row0 = wid * _ROWS_PER_W
    pltpu.sync_copy(model_hbm, table_v)

    idx = [idx0, idx1]
    val = [val0, val1]
    sin = [sin0, sin1]
    sout = [sout0, sout1]

    def start_in(g):
        r = row0 + g * _RCHUNK
        return pltpu.async_copy(pos_hbm.at[pl.ds(r, _RCHUNK), :], idx[g % 2],
                                sin[g % 2])

    def start_out(g):
        r = row0 + g * _RCHUNK
        return pltpu.async_copy(val[g % 2], out_hbm.at[pl.ds(r, _RCHUNK), :],
                                sout[g % 2])

    in_h = {0: start_in(0), 1: start_in(1)}
    out_h = {}
    for g in range(_NCHUNK):
        b = g % 2
        in_h[g].wait()
        if g >= 2:
            out_h[g - 2].wait()
        ib, vb = idx[b], val[b]

        @plsc.parallel_loop(0, _RCHUNK, unroll=2)
        def _(r, ib=ib, vb=vb):
            for off in _OFFS:
                x = ib[r, pl.ds(off, _LANES)]
                vb[r, pl.ds(off, _LANES)] = plsc.load_gather(table_v, [x])

        out_h[g] = start_out(g)
        if g + 2 < _NCHUNK:
            in_h[g + 2] = start_in(g + 2)
    out_h[_NCHUNK - 2].wait()
    out_h[_NCHUNK - 1].wait()


def kernel(positions, model):
    return _gather_kernel(positions, model)
